# baseline (speedup 1.0000x reference)
import jax
import jax.numpy as jnp
from jax import lax
from jax.experimental import pallas as pl
from jax.experimental.pallas import tpu as pltpu
from jax.experimental.pallas import tpu_sc as plsc

N_NODES = 10000
N_EDGES = 320000
D = 128
DA = 136

NC = 2
NS = 16
NW = NC * NS
EPW = N_EDGES // NW
C = 40
G = EPW // C
NBUF = 4


def _make_sc_seg_sum(W: int):
    out_type = jax.ShapeDtypeStruct((NC, N_NODES, W), jnp.float32)
    scratch = [
        pltpu.VMEM((G, C), jnp.int32),
        pltpu.VMEM((G, C), jnp.int32),
        [pltpu.VMEM((C, W), jnp.float32) for _ in range(NBUF)],
        pltpu.VMEM_SHARED((N_NODES, W), jnp.float32),
        [pltpu.SemaphoreType.DMA for _ in range(NBUF)],
        [pltpu.SemaphoreType.DMA for _ in range(NBUF)],
    ]
    mesh = plsc.VectorSubcoreMesh(core_axis_name="c", subcore_axis_name="s")

    zoff = sorted({j * 16 for j in range(W // 16)} | {W - 16})

    def body(h_hbm, src_hbm, dst_hbm, aggp_hbm, idx_s, idx_d, rows, acc,
             sems, ssems):
        cid = lax.axis_index("c")
        sid = lax.axis_index("s")
        wid = cid * NS + sid

        zero16 = jnp.zeros((16,), jnp.float32)

        def zrow(r, carry):
            for j in zoff:
                rows[0][r, pl.ds(j, 16)] = zero16
            return carry

        lax.fori_loop(0, C, zrow, 0)

        def zb(z, carry):
            pltpu.sync_copy(rows[0], acc.at[pl.ds(sid * 640 + z * C, C)])
            return carry

        @pl.when(sid < NS - 1)
        def _zero_big():
            lax.fori_loop(0, 640 // C, zb, 0)

        @pl.when(sid == NS - 1)
        def _zero_tail():
            lax.fori_loop(0, 400 // C, zb, 0)

        pltpu.sync_copy(src_hbm.at[wid], idx_s)
        pltpu.sync_copy(dst_hbm.at[wid], idx_d)
        plsc.subcore_barrier()

        LOOK = NBUF - 2
        for b in range(LOOK):
            pltpu.async_copy(h_hbm.at[idx_s.at[b]], rows[b], sems[b])

        def step(c, b):
            bp = (b + LOOK) % NBUF

            @pl.when(c + LOOK < G)
            def _prefetch():
                @pl.when(c >= NBUF - LOOK)
                def _reuse_wait():
                    pltpu.make_async_copy(
                        rows[bp], acc.at[idx_d.at[0]], ssems[bp]).wait()

                pltpu.async_copy(h_hbm.at[idx_s.at[c + LOOK]], rows[bp],
                                 sems[bp])

            pltpu.make_async_copy(h_hbm.at[idx_s.at[c]], rows[b],
                                  sems[b]).wait()
            pltpu.async_copy(rows[b], acc.at[idx_d.at[c]], ssems[b],
                             add=True)

        def rnd(r, carry):
            for b in range(NBUF):
                step(r * NBUF + b, b)
            return carry

        lax.fori_loop(0, G // NBUF, rnd, 0)
        for t in range(G - NBUF * (G // NBUF)):
            step(NBUF * (G // NBUF) + t, t)
        for k in range(NBUF):
            b = (G - NBUF + k) % NBUF
            pltpu.make_async_copy(rows[b], acc.at[idx_d.at[0]],
                                  ssems[b]).wait()
        plsc.subcore_barrier()

        @pl.when(sid < NS - 1)
        def _write_big():
            pltpu.sync_copy(acc.at[pl.ds(sid * 640, 640)],
                            aggp_hbm.at[cid, pl.ds(sid * 640, 640)])

        @pl.when(sid == NS - 1)
        def _write_tail():
            pltpu.sync_copy(acc.at[pl.ds((NS - 1) * 640, 400)],
                            aggp_hbm.at[cid, pl.ds((NS - 1) * 640, 400)])

    return pl.kernel(
        body, out_type=out_type, mesh=mesh, scratch_types=scratch,
        compiler_params=pltpu.CompilerParams(use_tc_tiling_on_sc=False))


_sc_seg_sum_aug = _make_sc_seg_sum(DA)
_sc_seg_sum = _make_sc_seg_sum(D)

BN = 1000


def _tc1_body(h_ref, aggp_ref, ws_ref, wn_ref, b_ref, out_ref, invd_ref):
    agg = aggp_ref[0] + aggp_ref[1]
    feat = agg[:, :D]
    deg = agg[:, D]
    inv = 1.0 / jnp.maximum(deg, 1.0)
    hn = feat * inv[:, None]
    out = (jnp.dot(h_ref[...], ws_ref[...],
                   preferred_element_type=jnp.float32)
           + jnp.dot(hn, wn_ref[...], preferred_element_type=jnp.float32)
           + b_ref[...])
    out_ref[...] = jnp.maximum(out, 0.0)
    invd_ref[...] = jnp.broadcast_to(inv[:, None], (BN, 8))


def _tc1(h, aggp, W_self, W_neigh, b):
    return pl.pallas_call(
        _tc1_body,
        grid=(N_NODES // BN,),
        in_specs=[
            pl.BlockSpec((BN, D), lambda i: (i, 0)),
            pl.BlockSpec((NC, BN, DA), lambda i: (0, i, 0)),
            pl.BlockSpec((D, D), lambda i: (0, 0)),
            pl.BlockSpec((D, D), lambda i: (0, 0)),
            pl.BlockSpec((1, D), lambda i: (0, 0)),
        ],
        out_specs=[
            pl.BlockSpec((BN, D), lambda i: (i, 0)),
            pl.BlockSpec((BN, 8), lambda i: (i, 0)),
        ],
        out_shape=[
            jax.ShapeDtypeStruct((N_NODES, D), jnp.float32),
            jax.ShapeDtypeStruct((N_NODES, 8), jnp.float32),
        ],
    )(h, aggp, W_self, W_neigh, b.reshape(1, D))


def _tc2_body(h_ref, aggp_ref, invd_ref, ws_ref, wn_ref, b_ref, out_ref):
    agg = aggp_ref[0] + aggp_ref[1]
    inv = invd_ref[:, 0]
    hn = agg * inv[:, None]
    out = (jnp.dot(h_ref[...], ws_ref[...],
                   preferred_element_type=jnp.float32)
           + jnp.dot(hn, wn_ref[...], preferred_element_type=jnp.float32)
           + b_ref[...])
    out_ref[...] = jnp.maximum(out, 0.0)


def _tc2(h, aggp, invd, W_self, W_neigh, b):
    return pl.pallas_call(
        _tc2_body,
        grid=(N_NODES // BN,),
        in_specs=[
            pl.BlockSpec((BN, D), lambda i: (i, 0)),
            pl.BlockSpec((NC, BN, D), lambda i: (0, i, 0)),
            pl.BlockSpec((BN, 8), lambda i: (i, 0)),
            pl.BlockSpec((D, D), lambda i: (0, 0)),
            pl.BlockSpec((D, D), lambda i: (0, 0)),
            pl.BlockSpec((1, D), lambda i: (0, 0)),
        ],
        out_specs=pl.BlockSpec((BN, D), lambda i: (i, 0)),
        out_shape=jax.ShapeDtypeStruct((N_NODES, D), jnp.float32),
    )(h, aggp, invd, W_self, W_neigh, b.reshape(1, D))


def kernel(in_feat, edge_index, W_self1, W_neigh1, b1, W_self2, W_neigh2,
           b2):
    edge_index = edge_index.astype(jnp.int32)
    src3 = edge_index[0].reshape(NW, G, C)
    dst3 = edge_index[1].reshape(NW, G, C)
    h = in_feat.astype(jnp.float32)
    haug = jnp.concatenate(
        [h, jnp.ones((N_NODES, 1), jnp.float32),
         jnp.zeros((N_NODES, DA - D - 1), jnp.float32)], axis=1)

    aggp1 = _sc_seg_sum_aug(haug, src3, dst3)
    h1, invd = _tc1(h, aggp1, W_self1, W_neigh1, b1)
    aggp2 = _sc_seg_sum(h1, src3, dst3)
    out = _tc2(h1, aggp2, invd, W_self2, W_neigh2, b2)
    return out


# speedup vs baseline: 1.0279x; 1.0279x over previous
import jax
import jax.numpy as jnp
from jax import lax
from jax.experimental import pallas as pl
from jax.experimental.pallas import tpu as pltpu
from jax.experimental.pallas import tpu_sc as plsc

N_NODES = 10000
N_EDGES = 320000
D = 128
DA = 136

NC = 2
NS = 16
NW = NC * NS
EPW = N_EDGES // NW
C = 40
G = EPW // C
NBUF = 4


def _make_sc_seg_sum(W: int):
    out_type = jax.ShapeDtypeStruct((NC, N_NODES, W), jnp.float32)
    scratch = [
        pltpu.VMEM((G, C), jnp.int32),
        pltpu.VMEM((G, C), jnp.int32),
        [pltpu.VMEM((C, W), jnp.float32) for _ in range(NBUF)],
        pltpu.VMEM_SHARED((N_NODES, W), jnp.float32),
        [pltpu.SemaphoreType.DMA for _ in range(NBUF)],
    ]
    mesh = plsc.VectorSubcoreMesh(core_axis_name="c", subcore_axis_name="s")

    zoff = sorted({j * 16 for j in range(W // 16)} | {W - 16})

    def body(h_hbm, src_hbm, dst_hbm, aggp_hbm, idx_s, idx_d, rows, acc,
             sems):
        cid = lax.axis_index("c")
        sid = lax.axis_index("s")
        wid = cid * NS + sid

        zero16 = jnp.zeros((16,), jnp.float32)

        def zrow(r, carry):
            for j in zoff:
                rows[0][r, pl.ds(j, 16)] = zero16
            return carry

        lax.fori_loop(0, C, zrow, 0)

        def zb(z, carry):
            pltpu.sync_copy(rows[0], acc.at[pl.ds(sid * 640 + z * C, C)])
            return carry

        @pl.when(sid < NS - 1)
        def _zero_big():
            lax.fori_loop(0, 640 // C, zb, 0)

        @pl.when(sid == NS - 1)
        def _zero_tail():
            lax.fori_loop(0, 400 // C, zb, 0)

        pltpu.sync_copy(src_hbm.at[wid], idx_s)
        pltpu.sync_copy(dst_hbm.at[wid], idx_d)
        plsc.subcore_barrier()

        for b in range(NBUF):
            pltpu.async_copy(h_hbm.at[idx_s.at[b]], rows[b], sems[b])

        def step(c, b):
            pltpu.make_async_copy(h_hbm.at[idx_s.at[c]], rows[b],
                                  sems[b]).wait()
            pltpu.sync_copy(rows[b], acc.at[idx_d.at[c]], add=True)

            @pl.when(c + NBUF < G)
            def _prefetch():
                pltpu.async_copy(h_hbm.at[idx_s.at[c + NBUF]], rows[b],
                                 sems[b])

        def rnd(r, carry):
            for b in range(NBUF):
                step(r * NBUF + b, b)
            return carry

        lax.fori_loop(0, G // NBUF, rnd, 0)
        for t in range(G - NBUF * (G // NBUF)):
            step(NBUF * (G // NBUF) + t, t)
        plsc.subcore_barrier()

        @pl.when(sid < NS - 1)
        def _write_big():
            pltpu.sync_copy(acc.at[pl.ds(sid * 640, 640)],
                            aggp_hbm.at[cid, pl.ds(sid * 640, 640)])

        @pl.when(sid == NS - 1)
        def _write_tail():
            pltpu.sync_copy(acc.at[pl.ds((NS - 1) * 640, 400)],
                            aggp_hbm.at[cid, pl.ds((NS - 1) * 640, 400)])

    return pl.kernel(
        body, out_type=out_type, mesh=mesh, scratch_types=scratch,
        compiler_params=pltpu.CompilerParams(use_tc_tiling_on_sc=False))


_sc_seg_sum_aug = _make_sc_seg_sum(DA)
_sc_seg_sum = _make_sc_seg_sum(D)

BN = 1000


def _tc_pre_body(x_ref, wn_ref, ws_ref, b_ref, p1_ref, s1_ref):
    xb = x_ref[...]
    p = jnp.dot(xb, wn_ref[...], preferred_element_type=jnp.float32)
    ones = jnp.ones((BN, 1), jnp.float32)
    zeros = jnp.zeros((BN, DA - D - 1), jnp.float32)
    p1_ref[...] = jnp.concatenate([p, ones, zeros], axis=1)
    s1_ref[...] = (jnp.dot(xb, ws_ref[...],
                           preferred_element_type=jnp.float32)
                   + b_ref[...])


def _tc_pre(x, W_neigh, W_self, b):
    return pl.pallas_call(
        _tc_pre_body,
        grid=(N_NODES // BN,),
        in_specs=[
            pl.BlockSpec((BN, D), lambda i: (i, 0)),
            pl.BlockSpec((D, D), lambda i: (0, 0)),
            pl.BlockSpec((D, D), lambda i: (0, 0)),
            pl.BlockSpec((1, D), lambda i: (0, 0)),
        ],
        out_specs=[
            pl.BlockSpec((BN, DA), lambda i: (i, 0)),
            pl.BlockSpec((BN, D), lambda i: (i, 0)),
        ],
        out_shape=[
            jax.ShapeDtypeStruct((N_NODES, DA), jnp.float32),
            jax.ShapeDtypeStruct((N_NODES, D), jnp.float32),
        ],
    )(x, W_neigh, W_self, b.reshape(1, D))


def _tc_mid_body(aggp_ref, s1_ref, wn_ref, ws_ref, b_ref, p2_ref, s2_ref,
                 invd_ref):
    agg = aggp_ref[0] + aggp_ref[1]
    inv = 1.0 / jnp.maximum(agg[:, D], 1.0)
    h1 = jnp.maximum(s1_ref[...] + agg[:, :D] * inv[:, None], 0.0)
    p2_ref[...] = jnp.dot(h1, wn_ref[...],
                          preferred_element_type=jnp.float32)
    s2_ref[...] = (jnp.dot(h1, ws_ref[...],
                           preferred_element_type=jnp.float32)
                   + b_ref[...])
    invd_ref[...] = jnp.broadcast_to(inv[:, None], (BN, 8))


def _tc_mid(aggp, s1, W_neigh, W_self, b):
    return pl.pallas_call(
        _tc_mid_body,
        grid=(N_NODES // BN,),
        in_specs=[
            pl.BlockSpec((NC, BN, DA), lambda i: (0, i, 0)),
            pl.BlockSpec((BN, D), lambda i: (i, 0)),
            pl.BlockSpec((D, D), lambda i: (0, 0)),
            pl.BlockSpec((D, D), lambda i: (0, 0)),
            pl.BlockSpec((1, D), lambda i: (0, 0)),
        ],
        out_specs=[
            pl.BlockSpec((BN, D), lambda i: (i, 0)),
            pl.BlockSpec((BN, D), lambda i: (i, 0)),
            pl.BlockSpec((BN, 8), lambda i: (i, 0)),
        ],
        out_shape=[
            jax.ShapeDtypeStruct((N_NODES, D), jnp.float32),
            jax.ShapeDtypeStruct((N_NODES, D), jnp.float32),
            jax.ShapeDtypeStruct((N_NODES, 8), jnp.float32),
        ],
    )(aggp, s1, W_neigh, W_self, b.reshape(1, D))


def _tc_post_body(aggp_ref, s2_ref, invd_ref, out_ref):
    agg = aggp_ref[0] + aggp_ref[1]
    inv = invd_ref[:, 0]
    out_ref[...] = jnp.maximum(s2_ref[...] + agg * inv[:, None], 0.0)


def _tc_post(aggp, s2, invd):
    return pl.pallas_call(
        _tc_post_body,
        grid=(N_NODES // BN,),
        in_specs=[
            pl.BlockSpec((NC, BN, D), lambda i: (0, i, 0)),
            pl.BlockSpec((BN, D), lambda i: (i, 0)),
            pl.BlockSpec((BN, 8), lambda i: (i, 0)),
        ],
        out_specs=pl.BlockSpec((BN, D), lambda i: (i, 0)),
        out_shape=jax.ShapeDtypeStruct((N_NODES, D), jnp.float32),
    )(aggp, s2, invd)


def kernel(in_feat, edge_index, W_self1, W_neigh1, b1, W_self2, W_neigh2,
           b2):
    edge_index = edge_index.astype(jnp.int32)
    src3 = edge_index[0].reshape(NW, G, C)
    dst3 = edge_index[1].reshape(NW, G, C)
    h = in_feat.astype(jnp.float32)

    p1aug, s1 = _tc_pre(h, W_neigh1, W_self1, b1)
    aggp1 = _sc_seg_sum_aug(p1aug, src3, dst3)
    p2, s2, invd = _tc_mid(aggp1, s1, W_neigh2, W_self2, b2)
    aggp2 = _sc_seg_sum(p2, src3, dst3)
    out = _tc_post(aggp2, s2, invd)
    return out
